# Initial kernel scaffold; baseline (speedup 1.0000x reference)
import jax
import jax.numpy as jnp
from jax.experimental import pallas as pl


def kernel(input_ids, weight):
    raise NotImplementedError("write your pallas kernel here")


# speedup vs baseline: 14.4641x; 14.4641x over previous
import functools

import jax
import jax.numpy as jnp
from jax import lax
from jax.experimental import pallas as pl
from jax.experimental.pallas import tpu as pltpu


def _gather_tile(ids_ref, w_hbm, out_ref, sem, *, unroll):
    tile_rows = out_ref.shape[0]
    base = pl.program_id(0) * tile_rows

    def issue(c, carry):
        r = c * unroll
        for u in range(unroll):
            pltpu.make_async_copy(
                w_hbm.at[ids_ref[base + r + u]], out_ref.at[r + u], sem
            ).start()
        return carry

    lax.fori_loop(0, tile_rows // unroll, issue, 0)

    pltpu.make_async_copy(w_hbm.at[pl.ds(0, tile_rows)], out_ref, sem).wait()


def kernel(input_ids: jax.Array, weight: jax.Array,
           *, tile_rows: int = 2048, unroll: int = 8) -> jax.Array:
    B, S = input_ids.shape
    V, D = weight.shape
    n_rows = B * S

    n_tiles = pl.cdiv(n_rows, tile_rows)
    n_pad = n_tiles * tile_rows

    ids_flat = input_ids.reshape(n_rows).astype(jnp.int32)
    if n_pad != n_rows:
        ids_flat = jnp.pad(ids_flat, (0, n_pad - n_rows))

    out = pl.pallas_call(
        functools.partial(_gather_tile, unroll=unroll),
        out_shape=jax.ShapeDtypeStruct((n_pad, D), weight.dtype),
        grid_spec=pltpu.PrefetchScalarGridSpec(
            num_scalar_prefetch=1,
            grid=(n_tiles,),
            in_specs=[pl.BlockSpec(memory_space=pl.ANY)],
            out_specs=pl.BlockSpec((tile_rows, D), lambda i, ids: (i, 0)),
            scratch_shapes=[pltpu.SemaphoreType.DMA],
        ),
        compiler_params=pltpu.CompilerParams(
            dimension_semantics=("parallel",),
            disable_bounds_checks=True,
        ),
    )(ids_flat, weight)

    if n_pad != n_rows:
        out = out[:n_rows]
    return out.reshape(B, S, D)
